# baseline (speedup 1.0000x reference)
import jax
import jax.numpy as jnp
from jax import lax
from jax.experimental import pallas as pl
from jax.experimental.pallas import tpu as pltpu
from jax.experimental.pallas import tpu_sc as plsc

N_NODES = 10000
D = 128
N_EDGES = 320000
NC = 2
NS = 16
NW = NC * NS
GROUP = 128
NG = 80
NGA = 80
NGB = 2 * NG - NGA
E_PAD = NW * NG * GROUP
ACC_ROWS = 10240
RPS = ACC_ROWS // NS
CROWS = ACC_ROWS // D


def _bincount_pair(srp_v, p, cnt_v):
    ones16 = jnp.ones((16,), jnp.float32)
    for j in range(GROUP // 16):
        idx = srp_v[p, 1, pl.ds(j * 16, 16)]
        row = lax.shift_right_logical(idx, 7)
        col = lax.bitwise_and(idx, 127)
        plsc.addupdate_scatter(cnt_v, [row, col], ones16)


def _sc_aggregate_body(x_hbm, sr_hbm, zx_hbm, acc_out, cnt_out,
                       srpa_v, srpb_v, rows0_v, rows1_v, cnt_v, iota_v,
                       acc_sh, cnt_sh, sem_g0, sem_g1, sem_s0, sem_s1):
    cid = lax.axis_index("c")
    sid = lax.axis_index("s")
    wid = sid * NC + cid
    rbase = sid * RPS
    gbase = jnp.where(cid == 0, sid * NGA, NS * NGA + sid * NGB)
    nb = jnp.where(cid == 0, NGA // 4, NGB // 4)

    def wait_g(rows_v, sem):
        pltpu.make_async_copy(x_hbm.at[srpa_v.at[0, 0]], rows_v, sem).wait()

    def wait_s(rows_v, sem):
        pltpu.make_async_copy(rows_v, acc_sh.at[srpa_v.at[0, 1]], sem).wait()

    pltpu.sync_copy(zx_hbm, acc_sh.at[pl.ds(rbase, RPS)])

    def z(i, c):
        for j in range(D // 16):
            cnt_v[i, pl.ds(j * 16, 16)] = jnp.zeros((16,), jnp.float32)
        return c

    lax.fori_loop(0, CROWS, z, 0)
    for j in range(CROWS // 16):
        iota_v[pl.ds(j * 16, 16)] = (
            lax.iota(jnp.int32, 16) + (16 * j))
    @pl.when(sid == 0)
    def _():
        pltpu.sync_copy(zx_hbm.at[pl.ds(0, CROWS)], cnt_sh)
    pltpu.sync_copy(sr_hbm.at[pl.ds(gbase, 2)], srpa_v)
    pltpu.async_copy(x_hbm.at[srpa_v.at[0, 0]], rows0_v, sem_g0)
    plsc.subcore_barrier()

    def body(k, carry):
        @pl.when(k > 0)
        def _():
            wait_s(rows1_v, sem_s1)
        pltpu.async_copy(x_hbm.at[srpa_v.at[1, 0]], rows1_v, sem_g1)
        wait_g(rows0_v, sem_g0)
        _bincount_pair(srpa_v, 0, cnt_v)
        pltpu.async_copy(rows0_v, acc_sh.at[srpa_v.at[0, 1]], sem_s0,
                         add=True)
        pltpu.sync_copy(sr_hbm.at[pl.ds(gbase + 4 * k + 2, 2)], srpb_v)
        wait_s(rows0_v, sem_s0)
        pltpu.async_copy(x_hbm.at[srpb_v.at[0, 0]], rows0_v, sem_g0)
        wait_g(rows1_v, sem_g1)
        _bincount_pair(srpa_v, 1, cnt_v)
        pltpu.async_copy(rows1_v, acc_sh.at[srpa_v.at[1, 1]], sem_s1,
                         add=True)
        wait_s(rows1_v, sem_s1)
        pltpu.async_copy(x_hbm.at[srpb_v.at[1, 0]], rows1_v, sem_g1)
        wait_g(rows0_v, sem_g0)
        _bincount_pair(srpb_v, 0, cnt_v)
        pltpu.async_copy(rows0_v, acc_sh.at[srpb_v.at[0, 1]], sem_s0,
                         add=True)

        @pl.when(k < nb - 1)
        def _():
            wait_s(rows0_v, sem_s0)
            pltpu.sync_copy(sr_hbm.at[pl.ds(gbase + 4 * k + 4, 2)], srpa_v)
            pltpu.async_copy(x_hbm.at[srpa_v.at[0, 0]], rows0_v, sem_g0)

        wait_g(rows1_v, sem_g1)
        _bincount_pair(srpb_v, 1, cnt_v)
        pltpu.async_copy(rows1_v, acc_sh.at[srpb_v.at[1, 1]], sem_s1,
                         add=True)
        return carry

    lax.fori_loop(0, nb, body, 0)
    wait_s(rows0_v, sem_s0)
    wait_s(rows1_v, sem_s1)
    pltpu.sync_copy(cnt_v, cnt_sh.at[iota_v], add=True)
    plsc.subcore_barrier()

    pltpu.sync_copy(acc_sh.at[pl.ds(rbase, RPS)],
                    acc_out.at[cid, pl.ds(rbase, RPS)])

    @pl.when(sid == 0)
    def _():
        pltpu.sync_copy(cnt_sh, cnt_out.at[cid])


def _tc_pre_body(x_ref, w1_ref, b_ref, o_ref):
    o_ref[...] = jnp.dot(
        x_ref[...], w1_ref[...],
        preferred_element_type=jnp.float32) + b_ref[...]


def _tc_combine_body(x_ref, pre_ref, acc_ref, c0_ref, c1_ref, w2_ref, o_ref):
    cnt = c0_ref[...] + c1_ref[...] + 1.0
    agg = (acc_ref[0] + acc_ref[1] + x_ref[...]) / cnt
    o_ref[...] = pre_ref[...] + jnp.dot(
        agg, w2_ref[...], preferred_element_type=jnp.float32)


def kernel(x, senders, receivers, n_nodes, W, b):
    pad = E_PAD - N_EDGES
    fill = jnp.arange(pad, dtype=jnp.int32)
    s_pad = jnp.concatenate([senders.astype(jnp.int32), fill % N_NODES])
    r_pad = jnp.concatenate(
        [receivers.astype(jnp.int32),
         N_NODES + fill % (ACC_ROWS - N_NODES)])
    sr = jnp.stack(
        [s_pad.reshape(NW * NG, GROUP), r_pad.reshape(NW * NG, GROUP)],
        axis=1)
    zx = jnp.zeros((RPS, D), jnp.float32)

    mesh = plsc.VectorSubcoreMesh(core_axis_name="c", subcore_axis_name="s")
    acc, cnt_flat = pl.kernel(
        _sc_aggregate_body,
        out_type=[
            jax.ShapeDtypeStruct((NC, ACC_ROWS, D), jnp.float32),
            jax.ShapeDtypeStruct((NC, CROWS, D), jnp.float32),
        ],
        mesh=mesh,
        compiler_params=pltpu.CompilerParams(needs_layout_passes=False),
        scratch_types=[
            pltpu.VMEM((2, 2, GROUP), jnp.int32),
            pltpu.VMEM((2, 2, GROUP), jnp.int32),
            pltpu.VMEM((GROUP, D), jnp.float32),
            pltpu.VMEM((GROUP, D), jnp.float32),
            pltpu.VMEM((CROWS, D), jnp.float32),
            pltpu.VMEM((CROWS,), jnp.int32),
            pltpu.VMEM_SHARED((ACC_ROWS, D), jnp.float32),
            pltpu.VMEM_SHARED((CROWS, D), jnp.float32),
            pltpu.SemaphoreType.DMA,
            pltpu.SemaphoreType.DMA,
            pltpu.SemaphoreType.DMA,
            pltpu.SemaphoreType.DMA,
        ],
    )(x, sr, zx)

    c0 = cnt_flat[0].reshape(ACC_ROWS)[:N_NODES, None]
    c1 = cnt_flat[1].reshape(ACC_ROWS)[:N_NODES, None]

    w1t = W[:, :D].T
    w2t = W[:, D:].T
    b2 = b.reshape(1, D)

    blk = 1000
    pre = pl.pallas_call(
        _tc_pre_body,
        grid=(N_NODES // blk,),
        in_specs=[
            pl.BlockSpec((blk, D), lambda i: (i, 0)),
            pl.BlockSpec((D, D), lambda i: (0, 0)),
            pl.BlockSpec((1, D), lambda i: (0, 0)),
        ],
        out_specs=pl.BlockSpec((blk, D), lambda i: (i, 0)),
        out_shape=jax.ShapeDtypeStruct((N_NODES, D), jnp.float32),
    )(x, w1t, b2)

    out = pl.pallas_call(
        _tc_combine_body,
        grid=(N_NODES // blk,),
        in_specs=[
            pl.BlockSpec((blk, D), lambda i: (i, 0)),
            pl.BlockSpec((blk, D), lambda i: (i, 0)),
            pl.BlockSpec((NC, blk, D), lambda i: (0, i, 0)),
            pl.BlockSpec((blk, 1), lambda i: (i, 0)),
            pl.BlockSpec((blk, 1), lambda i: (i, 0)),
            pl.BlockSpec((D, D), lambda i: (0, 0)),
        ],
        out_specs=pl.BlockSpec((blk, D), lambda i: (i, 0)),
        out_shape=jax.ShapeDtypeStruct((N_NODES, D), jnp.float32),
    )(x, pre, acc, c0, c1, w2t)
    return out


# speedup vs baseline: 1.0291x; 1.0291x over previous
import jax
import jax.numpy as jnp
from jax import lax
from jax.experimental import pallas as pl
from jax.experimental.pallas import tpu as pltpu
from jax.experimental.pallas import tpu_sc as plsc

N_NODES = 10000
D = 128
N_EDGES = 320000
NC = 2
NS = 16
NW = NC * NS
GROUP = 128
NG = 80
NGA = 80
NGB = 2 * NG - NGA
E_PAD = NW * NG * GROUP
ACC_ROWS = 10240
RPS = ACC_ROWS // NS


def _bincount_pair(srp_v, p, cnt_v):
    ones16 = jnp.ones((16,), jnp.float32)
    for j in range(GROUP // 16):
        idx = srp_v[p, 1, pl.ds(j * 16, 16)]
        plsc.addupdate_scatter(cnt_v, [idx], ones16)


def _sc_aggregate_body(x_hbm, sr_hbm, zx_hbm, acc_out, cnt_out,
                       srpa_v, srpb_v, rows0_v, rows1_v, cnt_v, acc_sh,
                       sem_g0, sem_g1, sem_s0, sem_s1):
    cid = lax.axis_index("c")
    sid = lax.axis_index("s")
    wid = sid * NC + cid
    rbase = sid * RPS
    gbase = jnp.where(cid == 0, sid * NGA, NS * NGA + sid * NGB)
    nb = jnp.where(cid == 0, NGA // 4, NGB // 4)

    def wait_g(rows_v, sem):
        pltpu.make_async_copy(x_hbm.at[srpa_v.at[0, 0]], rows_v, sem).wait()

    def wait_s(rows_v, sem):
        pltpu.make_async_copy(rows_v, acc_sh.at[srpa_v.at[0, 1]], sem).wait()

    pltpu.sync_copy(zx_hbm, acc_sh.at[pl.ds(rbase, RPS)])

    def z(i, c):
        cnt_v[pl.ds(i * 16, 16)] = jnp.zeros((16,), jnp.float32)
        return c

    lax.fori_loop(0, ACC_ROWS // 16, z, 0)
    pltpu.sync_copy(sr_hbm.at[pl.ds(gbase, 2)], srpa_v)
    pltpu.async_copy(x_hbm.at[srpa_v.at[0, 0]], rows0_v, sem_g0)
    plsc.subcore_barrier()

    def body(k, carry):
        @pl.when(k > 0)
        def _():
            wait_s(rows1_v, sem_s1)
        pltpu.async_copy(x_hbm.at[srpa_v.at[1, 0]], rows1_v, sem_g1)
        wait_g(rows0_v, sem_g0)
        _bincount_pair(srpa_v, 0, cnt_v)
        pltpu.async_copy(rows0_v, acc_sh.at[srpa_v.at[0, 1]], sem_s0,
                         add=True)
        pltpu.sync_copy(sr_hbm.at[pl.ds(gbase + 4 * k + 2, 2)], srpb_v)
        wait_s(rows0_v, sem_s0)
        pltpu.async_copy(x_hbm.at[srpb_v.at[0, 0]], rows0_v, sem_g0)
        wait_g(rows1_v, sem_g1)
        _bincount_pair(srpa_v, 1, cnt_v)
        pltpu.async_copy(rows1_v, acc_sh.at[srpa_v.at[1, 1]], sem_s1,
                         add=True)
        wait_s(rows1_v, sem_s1)
        pltpu.async_copy(x_hbm.at[srpb_v.at[1, 0]], rows1_v, sem_g1)
        wait_g(rows0_v, sem_g0)
        _bincount_pair(srpb_v, 0, cnt_v)
        pltpu.async_copy(rows0_v, acc_sh.at[srpb_v.at[0, 1]], sem_s0,
                         add=True)

        @pl.when(k < nb - 1)
        def _():
            wait_s(rows0_v, sem_s0)
            pltpu.sync_copy(sr_hbm.at[pl.ds(gbase + 4 * k + 4, 2)], srpa_v)
            pltpu.async_copy(x_hbm.at[srpa_v.at[0, 0]], rows0_v, sem_g0)

        wait_g(rows1_v, sem_g1)
        _bincount_pair(srpb_v, 1, cnt_v)
        pltpu.async_copy(rows1_v, acc_sh.at[srpb_v.at[1, 1]], sem_s1,
                         add=True)
        return carry

    lax.fori_loop(0, nb, body, 0)
    wait_s(rows0_v, sem_s0)
    wait_s(rows1_v, sem_s1)
    plsc.subcore_barrier()

    pltpu.sync_copy(acc_sh.at[pl.ds(rbase, RPS)],
                    acc_out.at[cid, pl.ds(rbase, RPS)])
    pltpu.sync_copy(cnt_v, cnt_out.at[pl.ds(wid * ACC_ROWS, ACC_ROWS)])


def _tc_pre_body(x_ref, w1_ref, b_ref, o_ref):
    o_ref[...] = jnp.dot(
        x_ref[...], w1_ref[...],
        preferred_element_type=jnp.float32) + b_ref[...]


def _tc_combine_body(x_ref, pre_ref, acc_ref, cnt_ref, w2_ref, o_ref):
    cnt = jnp.sum(cnt_ref[...], axis=1, keepdims=True) + 1.0
    agg = (acc_ref[0] + acc_ref[1] + x_ref[...]) / cnt
    o_ref[...] = pre_ref[...] + jnp.dot(
        agg, w2_ref[...], preferred_element_type=jnp.float32)


def kernel(x, senders, receivers, n_nodes, W, b):
    pad = E_PAD - N_EDGES
    fill = jnp.arange(pad, dtype=jnp.int32)
    s_pad = jnp.concatenate([senders.astype(jnp.int32), fill % N_NODES])
    r_pad = jnp.concatenate(
        [receivers.astype(jnp.int32),
         N_NODES + fill % (ACC_ROWS - N_NODES)])
    sr = jnp.stack(
        [s_pad.reshape(NW * NG, GROUP), r_pad.reshape(NW * NG, GROUP)],
        axis=1)
    zx = jnp.zeros((RPS, D), jnp.float32)

    mesh = plsc.VectorSubcoreMesh(core_axis_name="c", subcore_axis_name="s")
    acc, cnt_flat = pl.kernel(
        _sc_aggregate_body,
        out_type=[
            jax.ShapeDtypeStruct((NC, ACC_ROWS, D), jnp.float32),
            jax.ShapeDtypeStruct((NW * ACC_ROWS,), jnp.float32),
        ],
        mesh=mesh,
        compiler_params=pltpu.CompilerParams(needs_layout_passes=False),
        scratch_types=[
            pltpu.VMEM((2, 2, GROUP), jnp.int32),
            pltpu.VMEM((2, 2, GROUP), jnp.int32),
            pltpu.VMEM((GROUP, D), jnp.float32),
            pltpu.VMEM((GROUP, D), jnp.float32),
            pltpu.VMEM((ACC_ROWS,), jnp.float32),
            pltpu.VMEM_SHARED((ACC_ROWS, D), jnp.float32),
            pltpu.SemaphoreType.DMA,
            pltpu.SemaphoreType.DMA,
            pltpu.SemaphoreType.DMA,
            pltpu.SemaphoreType.DMA,
        ],
    )(x, sr, zx)

    cnt_cols = cnt_flat.reshape(NW, ACC_ROWS).T[:N_NODES]

    w1t = W[:, :D].T
    w2t = W[:, D:].T
    b2 = b.reshape(1, D)

    blk = 2000
    pre = pl.pallas_call(
        _tc_pre_body,
        grid=(N_NODES // blk,),
        in_specs=[
            pl.BlockSpec((blk, D), lambda i: (i, 0)),
            pl.BlockSpec((D, D), lambda i: (0, 0)),
            pl.BlockSpec((1, D), lambda i: (0, 0)),
        ],
        out_specs=pl.BlockSpec((blk, D), lambda i: (i, 0)),
        out_shape=jax.ShapeDtypeStruct((N_NODES, D), jnp.float32),
    )(x, w1t, b2)

    out = pl.pallas_call(
        _tc_combine_body,
        grid=(N_NODES // blk,),
        in_specs=[
            pl.BlockSpec((blk, D), lambda i: (i, 0)),
            pl.BlockSpec((blk, D), lambda i: (i, 0)),
            pl.BlockSpec((NC, blk, D), lambda i: (0, i, 0)),
            pl.BlockSpec((blk, NW), lambda i: (i, 0)),
            pl.BlockSpec((D, D), lambda i: (0, 0)),
        ],
        out_specs=pl.BlockSpec((blk, D), lambda i: (i, 0)),
        out_shape=jax.ShapeDtypeStruct((N_NODES, D), jnp.float32),
    )(x, pre, acc, cnt_cols, w2t)
    return out


# speedup vs baseline: 1.0514x; 1.0217x over previous
import jax
import jax.numpy as jnp
from jax import lax
from jax.experimental import pallas as pl
from jax.experimental.pallas import tpu as pltpu
from jax.experimental.pallas import tpu_sc as plsc

N_NODES = 10000
D = 128
N_EDGES = 320000
NC = 2
NS = 16
NW = NC * NS
GROUP = 128
NG = 80
NGA = 80
NGB = 2 * NG - NGA
E_PAD = NW * NG * GROUP
ACC_ROWS = 10240
RPS = ACC_ROWS // NS


def _bincount_pair(srp_v, p, cnt_v):
    ones16 = jnp.ones((16,), jnp.float32)
    for j in range(GROUP // 16):
        idx = srp_v[p, 1, pl.ds(j * 16, 16)]
        plsc.addupdate_scatter(cnt_v, [idx], ones16)


def _sc_aggregate_body(x_hbm, sr_hbm, zx_hbm, acc_out, cnt_out,
                       srpa_v, srpb_v, rows0_v, rows1_v, cnt_v, acc_sh,
                       sem_g0, sem_g1, sem_s0, sem_s1):
    cid = lax.axis_index("c")
    sid = lax.axis_index("s")
    wid = sid * NC + cid
    rbase = sid * RPS
    gbase = jnp.where(cid == 0, sid * NGA, NS * NGA + sid * NGB)
    nb = jnp.where(cid == 0, NGA // 4, NGB // 4)

    def wait_g(rows_v, sem):
        pltpu.make_async_copy(x_hbm.at[srpa_v.at[0, 0]], rows_v, sem).wait()

    def wait_s(rows_v, sem):
        pltpu.make_async_copy(rows_v, acc_sh.at[srpa_v.at[0, 1]], sem).wait()

    pltpu.sync_copy(zx_hbm, acc_sh.at[pl.ds(rbase, RPS)])

    def z(i, c):
        cnt_v[pl.ds(i * 16, 16)] = jnp.zeros((16,), jnp.float32)
        return c

    lax.fori_loop(0, ACC_ROWS // 16, z, 0)
    pltpu.sync_copy(sr_hbm.at[pl.ds(gbase, 2)], srpa_v)
    pltpu.async_copy(x_hbm.at[srpa_v.at[0, 0]], rows0_v, sem_g0)
    plsc.subcore_barrier()

    def body(k, carry):
        @pl.when(k > 0)
        def _():
            wait_s(rows1_v, sem_s1)
        pltpu.async_copy(x_hbm.at[srpa_v.at[1, 0]], rows1_v, sem_g1)
        wait_g(rows0_v, sem_g0)
        _bincount_pair(srpa_v, 0, cnt_v)
        pltpu.async_copy(rows0_v, acc_sh.at[srpa_v.at[0, 1]], sem_s0,
                         add=True)
        pltpu.sync_copy(sr_hbm.at[pl.ds(gbase + 4 * k + 2, 2)], srpb_v)
        wait_s(rows0_v, sem_s0)
        pltpu.async_copy(x_hbm.at[srpb_v.at[0, 0]], rows0_v, sem_g0)
        wait_g(rows1_v, sem_g1)
        _bincount_pair(srpa_v, 1, cnt_v)
        pltpu.async_copy(rows1_v, acc_sh.at[srpa_v.at[1, 1]], sem_s1,
                         add=True)
        wait_s(rows1_v, sem_s1)
        pltpu.async_copy(x_hbm.at[srpb_v.at[1, 0]], rows1_v, sem_g1)
        wait_g(rows0_v, sem_g0)
        _bincount_pair(srpb_v, 0, cnt_v)
        pltpu.async_copy(rows0_v, acc_sh.at[srpb_v.at[0, 1]], sem_s0,
                         add=True)

        @pl.when(k < nb - 1)
        def _():
            wait_s(rows0_v, sem_s0)
            pltpu.sync_copy(sr_hbm.at[pl.ds(gbase + 4 * k + 4, 2)], srpa_v)
            pltpu.async_copy(x_hbm.at[srpa_v.at[0, 0]], rows0_v, sem_g0)

        wait_g(rows1_v, sem_g1)
        _bincount_pair(srpb_v, 1, cnt_v)
        pltpu.async_copy(rows1_v, acc_sh.at[srpb_v.at[1, 1]], sem_s1,
                         add=True)
        return carry

    lax.fori_loop(0, nb, body, 0)
    wait_s(rows0_v, sem_s0)
    wait_s(rows1_v, sem_s1)
    plsc.subcore_barrier()

    pltpu.sync_copy(acc_sh.at[pl.ds(rbase, RPS)],
                    acc_out.at[cid, pl.ds(rbase, RPS)])
    pltpu.sync_copy(cnt_v, cnt_out.at[pl.ds(wid * ACC_ROWS, ACC_ROWS)])


def _tc_pre_body(x_ref, w1_ref, b_ref, o_ref):
    o_ref[...] = jnp.dot(
        x_ref[...], w1_ref[...],
        preferred_element_type=jnp.float32) + b_ref[...]


def _tc_combine_body(x_ref, pre_ref, acc_ref, cnt_ref, w2_ref, o_ref):
    cnt = jnp.sum(cnt_ref[...], axis=1, keepdims=True) + 1.0
    agg = (acc_ref[0] + acc_ref[1] + x_ref[...]) / cnt
    o_ref[...] = pre_ref[...] + jnp.dot(
        agg, w2_ref[...], preferred_element_type=jnp.float32)


def kernel(x, senders, receivers, n_nodes, W, b):
    pad = E_PAD - N_EDGES
    fill = jnp.arange(pad, dtype=jnp.int32)
    s_pad = jnp.concatenate([senders.astype(jnp.int32), fill % N_NODES])
    r_pad = jnp.concatenate(
        [receivers.astype(jnp.int32),
         N_NODES + fill % (ACC_ROWS - N_NODES)])
    sr = jnp.stack(
        [s_pad.reshape(NW * NG, GROUP), r_pad.reshape(NW * NG, GROUP)],
        axis=1)
    zx = jnp.zeros((RPS, D), jnp.float32)

    mesh = plsc.VectorSubcoreMesh(core_axis_name="c", subcore_axis_name="s")
    acc, cnt_flat = pl.kernel(
        _sc_aggregate_body,
        out_type=[
            jax.ShapeDtypeStruct((NC, ACC_ROWS, D), jnp.float32),
            jax.ShapeDtypeStruct((NW * ACC_ROWS,), jnp.float32),
        ],
        mesh=mesh,
        compiler_params=pltpu.CompilerParams(needs_layout_passes=False),
        scratch_types=[
            pltpu.VMEM((2, 2, GROUP), jnp.int32),
            pltpu.VMEM((2, 2, GROUP), jnp.int32),
            pltpu.VMEM((GROUP, D), jnp.float32),
            pltpu.VMEM((GROUP, D), jnp.float32),
            pltpu.VMEM((ACC_ROWS,), jnp.float32),
            pltpu.VMEM_SHARED((ACC_ROWS, D), jnp.float32),
            pltpu.SemaphoreType.DMA,
            pltpu.SemaphoreType.DMA,
            pltpu.SemaphoreType.DMA,
            pltpu.SemaphoreType.DMA,
        ],
    )(x, sr, zx)

    cnt_cols = cnt_flat.reshape(NW, ACC_ROWS).T

    w1t = W[:, :D].T
    w2t = W[:, D:].T
    b2 = b.reshape(1, D)

    blk = 2000
    pre = pl.pallas_call(
        _tc_pre_body,
        grid=(N_NODES // blk,),
        in_specs=[
            pl.BlockSpec((blk, D), lambda i: (i, 0)),
            pl.BlockSpec((D, D), lambda i: (0, 0)),
            pl.BlockSpec((1, D), lambda i: (0, 0)),
        ],
        out_specs=pl.BlockSpec((blk, D), lambda i: (i, 0)),
        out_shape=jax.ShapeDtypeStruct((N_NODES, D), jnp.float32),
    )(x, w1t, b2)

    out = pl.pallas_call(
        _tc_combine_body,
        grid=(N_NODES // blk,),
        in_specs=[
            pl.BlockSpec((blk, D), lambda i: (i, 0)),
            pl.BlockSpec((blk, D), lambda i: (i, 0)),
            pl.BlockSpec((NC, blk, D), lambda i: (0, i, 0)),
            pl.BlockSpec((blk, NW), lambda i: (i, 0)),
            pl.BlockSpec((D, D), lambda i: (0, 0)),
        ],
        out_specs=pl.BlockSpec((blk, D), lambda i: (i, 0)),
        out_shape=jax.ShapeDtypeStruct((N_NODES, D), jnp.float32),
    )(x, pre, acc, cnt_cols, w2t)
    return out


# speedup vs baseline: 1.0692x; 1.0169x over previous
import jax
import jax.numpy as jnp
from jax import lax
from jax.experimental import pallas as pl
from jax.experimental.pallas import tpu as pltpu
from jax.experimental.pallas import tpu_sc as plsc

N_NODES = 10000
D = 128
N_EDGES = 320000
NC = 2
NS = 16
NW = NC * NS
GROUP = 128
NG = 80
NGA = 80
NGB = 2 * NG - NGA
E_PAD = NW * NG * GROUP
ACC_ROWS = 10240
RPS = ACC_ROWS // NS


def _bincount_pair(srp_v, p, cnt_v):
    ones16 = jnp.ones((16,), jnp.float32)
    for j in range(GROUP // 16):
        idx = srp_v[p, 1, pl.ds(j * 16, 16)]
        plsc.addupdate_scatter(cnt_v, [idx], ones16)


def _sc_aggregate_body(x_hbm, sr_hbm, zx_hbm, acc_out, cnt_out,
                       srpa_v, srpb_v, rows0_v, rows1_v, cnt_v, acc_sh,
                       sem_g0, sem_g1, sem_s0, sem_s1):
    cid = lax.axis_index("c")
    sid = lax.axis_index("s")
    wid = sid * NC + cid
    rbase = sid * RPS
    gbase = jnp.where(cid == 0, sid * NGA, NS * NGA + sid * NGB)
    nb = jnp.where(cid == 0, NGA // 4, NGB // 4)

    def wait_g(rows_v, sem):
        pltpu.make_async_copy(x_hbm.at[srpa_v.at[0, 0]], rows_v, sem).wait()

    def wait_s(rows_v, sem):
        pltpu.make_async_copy(rows_v, acc_sh.at[srpa_v.at[0, 1]], sem).wait()

    pltpu.sync_copy(zx_hbm, acc_sh.at[pl.ds(rbase, RPS)])

    def z(i, c):
        cnt_v[pl.ds(i * 16, 16)] = jnp.zeros((16,), jnp.float32)
        return c

    lax.fori_loop(0, ACC_ROWS // 16, z, 0)
    pltpu.sync_copy(sr_hbm.at[pl.ds(gbase, 2)], srpa_v)
    pltpu.async_copy(x_hbm.at[srpa_v.at[0, 0]], rows0_v, sem_g0)
    plsc.subcore_barrier()

    def body(k, carry):
        @pl.when(k > 0)
        def _():
            wait_s(rows1_v, sem_s1)
        pltpu.async_copy(x_hbm.at[srpa_v.at[1, 0]], rows1_v, sem_g1)
        wait_g(rows0_v, sem_g0)
        _bincount_pair(srpa_v, 0, cnt_v)
        pltpu.async_copy(rows0_v, acc_sh.at[srpa_v.at[0, 1]], sem_s0,
                         add=True)
        pltpu.sync_copy(sr_hbm.at[pl.ds(gbase + 4 * k + 2, 2)], srpb_v)
        wait_s(rows0_v, sem_s0)
        pltpu.async_copy(x_hbm.at[srpb_v.at[0, 0]], rows0_v, sem_g0)
        wait_g(rows1_v, sem_g1)
        _bincount_pair(srpa_v, 1, cnt_v)
        pltpu.async_copy(rows1_v, acc_sh.at[srpa_v.at[1, 1]], sem_s1,
                         add=True)
        wait_s(rows1_v, sem_s1)
        pltpu.async_copy(x_hbm.at[srpb_v.at[1, 0]], rows1_v, sem_g1)
        wait_g(rows0_v, sem_g0)
        _bincount_pair(srpb_v, 0, cnt_v)
        pltpu.async_copy(rows0_v, acc_sh.at[srpb_v.at[0, 1]], sem_s0,
                         add=True)

        @pl.when(k < nb - 1)
        def _():
            wait_s(rows0_v, sem_s0)
            pltpu.sync_copy(sr_hbm.at[pl.ds(gbase + 4 * k + 4, 2)], srpa_v)
            pltpu.async_copy(x_hbm.at[srpa_v.at[0, 0]], rows0_v, sem_g0)

        wait_g(rows1_v, sem_g1)
        _bincount_pair(srpb_v, 1, cnt_v)
        pltpu.async_copy(rows1_v, acc_sh.at[srpb_v.at[1, 1]], sem_s1,
                         add=True)
        return carry

    lax.fori_loop(0, nb, body, 0)
    wait_s(rows0_v, sem_s0)
    wait_s(rows1_v, sem_s1)
    plsc.subcore_barrier()

    pltpu.sync_copy(acc_sh.at[pl.ds(rbase, RPS)],
                    acc_out.at[cid, pl.ds(rbase, RPS)])
    pltpu.sync_copy(cnt_v, cnt_out.at[pl.ds(wid * ACC_ROWS, ACC_ROWS)])


def _tc_pre_body(x_ref, w1_ref, b_ref, o_ref):
    o_ref[...] = jnp.dot(
        x_ref[...], w1_ref[...],
        preferred_element_type=jnp.float32) + b_ref[...]


def _tc_combine_body(x_ref, pre_ref, acc_ref, cnt_ref, w2_ref, o_ref):
    cnt = lax.dot_general(
        cnt_ref[...], jnp.ones((NW, 1), jnp.float32),
        (((0,), (0,)), ((), ())),
        preferred_element_type=jnp.float32) + 1.0
    agg = (acc_ref[0] + acc_ref[1] + x_ref[...]) / cnt
    o_ref[...] = pre_ref[...] + jnp.dot(
        agg, w2_ref[...], preferred_element_type=jnp.float32)


def kernel(x, senders, receivers, n_nodes, W, b):
    pad = E_PAD - N_EDGES
    fill = jnp.arange(pad, dtype=jnp.int32)
    s_pad = jnp.concatenate([senders.astype(jnp.int32), fill % N_NODES])
    r_pad = jnp.concatenate(
        [receivers.astype(jnp.int32),
         N_NODES + fill % (ACC_ROWS - N_NODES)])
    sr = jnp.stack(
        [s_pad.reshape(NW * NG, GROUP), r_pad.reshape(NW * NG, GROUP)],
        axis=1)
    zx = jnp.zeros((RPS, D), jnp.float32)

    mesh = plsc.VectorSubcoreMesh(core_axis_name="c", subcore_axis_name="s")
    acc, cnt_flat = pl.kernel(
        _sc_aggregate_body,
        out_type=[
            jax.ShapeDtypeStruct((NC, ACC_ROWS, D), jnp.float32),
            jax.ShapeDtypeStruct((NW * ACC_ROWS,), jnp.float32),
        ],
        mesh=mesh,
        compiler_params=pltpu.CompilerParams(needs_layout_passes=False),
        scratch_types=[
            pltpu.VMEM((2, 2, GROUP), jnp.int32),
            pltpu.VMEM((2, 2, GROUP), jnp.int32),
            pltpu.VMEM((GROUP, D), jnp.float32),
            pltpu.VMEM((GROUP, D), jnp.float32),
            pltpu.VMEM((ACC_ROWS,), jnp.float32),
            pltpu.VMEM_SHARED((ACC_ROWS, D), jnp.float32),
            pltpu.SemaphoreType.DMA,
            pltpu.SemaphoreType.DMA,
            pltpu.SemaphoreType.DMA,
            pltpu.SemaphoreType.DMA,
        ],
    )(x, sr, zx)

    cnt32 = cnt_flat.reshape(NW, ACC_ROWS)

    w1t = W[:, :D].T
    w2t = W[:, D:].T
    b2 = b.reshape(1, D)

    blk = 2048
    pre = pl.pallas_call(
        _tc_pre_body,
        grid=(pl.cdiv(N_NODES, blk),),
        in_specs=[
            pl.BlockSpec((blk, D), lambda i: (i, 0)),
            pl.BlockSpec((D, D), lambda i: (0, 0)),
            pl.BlockSpec((1, D), lambda i: (0, 0)),
        ],
        out_specs=pl.BlockSpec((blk, D), lambda i: (i, 0)),
        out_shape=jax.ShapeDtypeStruct((N_NODES, D), jnp.float32),
    )(x, w1t, b2)

    out = pl.pallas_call(
        _tc_combine_body,
        grid=(pl.cdiv(N_NODES, blk),),
        in_specs=[
            pl.BlockSpec((blk, D), lambda i: (i, 0)),
            pl.BlockSpec((blk, D), lambda i: (i, 0)),
            pl.BlockSpec((NC, blk, D), lambda i: (0, i, 0)),
            pl.BlockSpec((NW, blk), lambda i: (0, i)),
            pl.BlockSpec((D, D), lambda i: (0, 0)),
        ],
        out_specs=pl.BlockSpec((blk, D), lambda i: (i, 0)),
        out_shape=jax.ShapeDtypeStruct((N_NODES, D), jnp.float32),
    )(x, pre, acc, cnt32, w2t)
    return out
